# baseline (speedup 1.0000x reference)
import functools

import jax
import jax.numpy as jnp
from jax import lax
from jax.experimental import pallas as pl
from jax.experimental.pallas import tpu as pltpu
from jax.experimental.pallas import tpu_sc as plsc

PATHS = 64
N_TOK = 8192
D = 4096
NC = 2
NS = 16
L = 16
NW = NC * NS
TPW = N_TOK // NW
G = TPW // L
CH = 4
NCH = TPW // CH

_mesh = plsc.VectorSubcoreMesh(core_axis_name="c", subcore_axis_name="s")
_cparams = pltpu.CompilerParams(needs_layout_passes=False)


@functools.partial(
    pl.kernel,
    mesh=_mesh,
    out_type=(
        jax.ShapeDtypeStruct((N_TOK,), jnp.int32),
        jax.ShapeDtypeStruct((N_TOK,), jnp.int32),
        jax.ShapeDtypeStruct((NW * PATHS,), jnp.int32),
    ),
    scratch_types=[
        pltpu.VMEM((TPW * PATHS,), jnp.float32),
        pltpu.VMEM((TPW,), jnp.int32),
        pltpu.VMEM((TPW,), jnp.int32),
        pltpu.VMEM((PATHS,), jnp.int32),
    ],
    compiler_params=_cparams,
)
def _route_kernel(score_hbm, pid_hbm, rank_hbm, hist_hbm,
                  score_v, pid_v, rank_v, hist_v):
    wid = lax.axis_index("s") * NC + lax.axis_index("c")
    base = wid * TPW
    pltpu.sync_copy(score_hbm.at[pl.ds(base * PATHS, TPW * PATHS)], score_v)

    zeros = jnp.zeros((L,), jnp.int32)
    for p0 in range(0, PATHS, L):
        hist_v[pl.ds(p0, L)] = zeros
    lane = lax.iota(jnp.int32, L)

    def group_body(g, carry):
        rowb = (lane + g * L) * PATHS
        chains = []
        for k in range(4):
            best = plsc.load_gather(score_v, [rowb + k])
            bestp = jnp.full((L,), k, jnp.int32)
            chains.append([best, bestp])
        for p in range(4, PATHS):
            k = p % 4
            v = plsc.load_gather(score_v, [rowb + p])
            upd = v > chains[k][0]
            chains[k][0] = jnp.where(upd, v, chains[k][0])
            chains[k][1] = jnp.where(upd, jnp.full((L,), p, jnp.int32),
                                     chains[k][1])

        def merge(a, b):
            take_b = (b[0] > a[0]) | ((b[0] == a[0]) & (b[1] < a[1]))
            return [jnp.where(take_b, b[0], a[0]),
                    jnp.where(take_b, b[1], a[1])]

        m0 = merge(chains[0], chains[1])
        m1 = merge(chains[2], chains[3])
        best, bestp = merge(m0, m1)
        rank = jnp.zeros((L,), jnp.int32)
        later = jnp.zeros((L,), jnp.bool_)
        for s in range(1, L):
            prev = bestp.at[jnp.maximum(lane - s, 0)].get(
                mode="promise_in_bounds")
            rank = rank + jnp.where((lane >= s) & (prev == bestp), 1, 0)
            nxt = bestp.at[jnp.minimum(lane + s, L - 1)].get(
                mode="promise_in_bounds")
            later = later | ((lane < L - s) & (nxt == bestp))
        before = plsc.load_gather(hist_v, [bestp])
        pid_v[pl.ds(g * L, L)] = bestp
        rank_v[pl.ds(g * L, L)] = before + rank
        plsc.store_scatter(hist_v, [bestp], before + rank + 1,
                           mask=jnp.logical_not(later))
        return carry

    lax.fori_loop(0, G, group_body, 0)
    pltpu.sync_copy(pid_v, pid_hbm.at[pl.ds(base, TPW)])
    pltpu.sync_copy(rank_v, rank_hbm.at[pl.ds(base, TPW)])
    pltpu.sync_copy(hist_v, hist_hbm.at[pl.ds(wid * PATHS, PATHS)])


@functools.partial(
    pl.kernel,
    mesh=_mesh,
    out_type=(
        jax.ShapeDtypeStruct((N_TOK, D), jnp.float32),
        jax.ShapeDtypeStruct((PATHS,), jnp.int32),
    ),
    scratch_types=[
        pltpu.VMEM((NW * PATHS,), jnp.int32),
        pltpu.VMEM((TPW,), jnp.int32),
        pltpu.VMEM((TPW,), jnp.int32),
        pltpu.VMEM((PATHS,), jnp.int32),
        pltpu.VMEM((PATHS,), jnp.int32),
        pltpu.VMEM((NCH, CH), jnp.int32),
        pltpu.VMEM((CH, D), jnp.float32),
        pltpu.VMEM((CH, D), jnp.float32),
        pltpu.VMEM((CH, D), jnp.float32),
        pltpu.VMEM((CH, D), jnp.float32),
        pltpu.SemaphoreType.DMA,
        pltpu.SemaphoreType.DMA,
    ],
    compiler_params=_cparams,
)
def _dispatch_kernel(x_hbm, pid_hbm, rank_hbm, hist_hbm, out_hbm, cnt_hbm,
                     hist_all, pid_sl, rank_sl, base_v, cnt_v, dest2,
                     rows_a, rows_b, rows_c, rows_d, ld_sem, st_sem):
    wid = lax.axis_index("s") * NC + lax.axis_index("c")
    base = wid * TPW
    pltpu.sync_copy(hist_hbm, hist_all)
    pltpu.sync_copy(pid_hbm.at[pl.ds(base, TPW)], pid_sl)
    pltpu.sync_copy(rank_hbm.at[pl.ds(base, TPW)], rank_sl)

    zeros = jnp.zeros((L,), jnp.int32)
    for p0 in range(0, PATHS, L):
        tot = zeros
        prior = zeros
        for w in range(NW):
            h = hist_all[pl.ds(w * PATHS + p0, L)]
            tot = tot + h
            prior = prior + jnp.where(
                jnp.broadcast_to(w < wid, (L,)), h, zeros)
        cnt_v[pl.ds(p0, L)] = tot
        base_v[pl.ds(p0, L)] = prior

    carry = jnp.int32(0)
    for p0 in range(0, PATHS, L):
        t = cnt_v[pl.ds(p0, L)]
        excl = plsc.cumsum(t) - t + carry
        base_v[pl.ds(p0, L)] = base_v[pl.ds(p0, L)] + excl
        carry = carry + jnp.sum(t)

    lane = lax.iota(jnp.int32, L)

    def grp(g, c):
        pid_g = pid_sl[pl.ds(g * L, L)]
        rk = rank_sl[pl.ds(g * L, L)]
        db = plsc.load_gather(base_v, [pid_g])
        plsc.store_scatter(dest2, [(L // CH) * g + lane // CH,
                                   lane & (CH - 1)], db + rk)
        return c

    lax.fori_loop(0, G, grp, 0)

    @pl.when(wid == 0)
    def _():
        pltpu.sync_copy(cnt_v, cnt_hbm)

    def chunk_src(c):
        return x_hbm.at[pl.ds(base + c * CH, CH)]

    NBUF = 4
    bufs = (rows_a, rows_b, rows_c, rows_d)
    ld = [None] * NBUF
    st = [None] * NBUF
    for c in range(NBUF - 1):
        ld[c] = pltpu.async_copy(chunk_src(c), bufs[c], ld_sem)
    for c in range(NCH):
        b = c % NBUF
        ld[b].wait()
        st[b] = pltpu.async_copy(bufs[b], out_hbm.at[dest2.at[c]], st_sem)
        nc = c + NBUF - 1
        if nc < NCH:
            nb = nc % NBUF
            if st[nb] is not None:
                st[nb].wait()
            ld[nb] = pltpu.async_copy(chunk_src(nc), bufs[nb], ld_sem)
    for i in range(NBUF):
        st[(NCH - 1 - i) % NBUF].wait()


def kernel(inputs):
    n = inputs.shape[0]
    score = jax.random.normal(jax.random.key(1), (n * PATHS,),
                              dtype=jnp.float32)
    pid, rank, hist = _route_kernel(score)
    dispatched, counts = _dispatch_kernel(inputs, pid, rank, hist)
    return dispatched, pid, counts


# speedup vs baseline: 1.0080x; 1.0080x over previous
import functools

import jax
import jax.numpy as jnp
from jax import lax
from jax.experimental import pallas as pl
from jax.experimental.pallas import tpu as pltpu
from jax.experimental.pallas import tpu_sc as plsc

PATHS = 64
N_TOK = 8192
D = 4096
NC = 2
NS = 16
L = 16
NW = NC * NS
TPW = N_TOK // NW
G = TPW // L
CH = 8
NCH = TPW // CH

_mesh = plsc.VectorSubcoreMesh(core_axis_name="c", subcore_axis_name="s")
_cparams = pltpu.CompilerParams(needs_layout_passes=False)


@functools.partial(
    pl.kernel,
    mesh=_mesh,
    out_type=(
        jax.ShapeDtypeStruct((N_TOK,), jnp.int32),
        jax.ShapeDtypeStruct((N_TOK,), jnp.int32),
        jax.ShapeDtypeStruct((NW * PATHS,), jnp.int32),
    ),
    scratch_types=[
        pltpu.VMEM((TPW * PATHS,), jnp.float32),
        pltpu.VMEM((TPW,), jnp.int32),
        pltpu.VMEM((TPW,), jnp.int32),
        pltpu.VMEM((PATHS,), jnp.int32),
    ],
    compiler_params=_cparams,
)
def _route_kernel(score_hbm, pid_hbm, rank_hbm, hist_hbm,
                  score_v, pid_v, rank_v, hist_v):
    wid = lax.axis_index("s") * NC + lax.axis_index("c")
    base = wid * TPW
    pltpu.sync_copy(score_hbm.at[pl.ds(base * PATHS, TPW * PATHS)], score_v)

    zeros = jnp.zeros((L,), jnp.int32)
    for p0 in range(0, PATHS, L):
        hist_v[pl.ds(p0, L)] = zeros
    lane = lax.iota(jnp.int32, L)

    def group_body(g, carry):
        rowb = (lane + g * L) * PATHS
        chains = []
        for k in range(4):
            best = plsc.load_gather(score_v, [rowb + k])
            bestp = jnp.full((L,), k, jnp.int32)
            chains.append([best, bestp])
        for p in range(4, PATHS):
            k = p % 4
            v = plsc.load_gather(score_v, [rowb + p])
            upd = v > chains[k][0]
            chains[k][0] = jnp.where(upd, v, chains[k][0])
            chains[k][1] = jnp.where(upd, jnp.full((L,), p, jnp.int32),
                                     chains[k][1])

        def merge(a, b):
            take_b = (b[0] > a[0]) | ((b[0] == a[0]) & (b[1] < a[1]))
            return [jnp.where(take_b, b[0], a[0]),
                    jnp.where(take_b, b[1], a[1])]

        m0 = merge(chains[0], chains[1])
        m1 = merge(chains[2], chains[3])
        best, bestp = merge(m0, m1)
        rank = jnp.zeros((L,), jnp.int32)
        later = jnp.zeros((L,), jnp.bool_)
        for s in range(1, L):
            prev = bestp.at[jnp.maximum(lane - s, 0)].get(
                mode="promise_in_bounds")
            rank = rank + jnp.where((lane >= s) & (prev == bestp), 1, 0)
            nxt = bestp.at[jnp.minimum(lane + s, L - 1)].get(
                mode="promise_in_bounds")
            later = later | ((lane < L - s) & (nxt == bestp))
        before = plsc.load_gather(hist_v, [bestp])
        pid_v[pl.ds(g * L, L)] = bestp
        rank_v[pl.ds(g * L, L)] = before + rank
        plsc.store_scatter(hist_v, [bestp], before + rank + 1,
                           mask=jnp.logical_not(later))
        return carry

    lax.fori_loop(0, G, group_body, 0)
    pltpu.sync_copy(pid_v, pid_hbm.at[pl.ds(base, TPW)])
    pltpu.sync_copy(rank_v, rank_hbm.at[pl.ds(base, TPW)])
    pltpu.sync_copy(hist_v, hist_hbm.at[pl.ds(wid * PATHS, PATHS)])


@functools.partial(
    pl.kernel,
    mesh=_mesh,
    out_type=(
        jax.ShapeDtypeStruct((N_TOK, D), jnp.float32),
        jax.ShapeDtypeStruct((PATHS,), jnp.int32),
    ),
    scratch_types=[
        pltpu.VMEM((NW * PATHS,), jnp.int32),
        pltpu.VMEM((TPW,), jnp.int32),
        pltpu.VMEM((TPW,), jnp.int32),
        pltpu.VMEM((PATHS,), jnp.int32),
        pltpu.VMEM((PATHS,), jnp.int32),
        pltpu.VMEM((NCH, CH), jnp.int32),
        pltpu.VMEM((CH, D), jnp.float32),
        pltpu.VMEM((CH, D), jnp.float32),
        pltpu.VMEM((CH, D), jnp.float32),
        pltpu.SemaphoreType.DMA,
        pltpu.SemaphoreType.DMA,
    ],
    compiler_params=_cparams,
)
def _dispatch_kernel(x_hbm, pid_hbm, rank_hbm, hist_hbm, out_hbm, cnt_hbm,
                     hist_all, pid_sl, rank_sl, base_v, cnt_v, dest2,
                     rows_a, rows_b, rows_c, ld_sem, st_sem):
    wid = lax.axis_index("s") * NC + lax.axis_index("c")
    base = wid * TPW
    pltpu.sync_copy(hist_hbm, hist_all)
    pltpu.sync_copy(pid_hbm.at[pl.ds(base, TPW)], pid_sl)
    pltpu.sync_copy(rank_hbm.at[pl.ds(base, TPW)], rank_sl)

    zeros = jnp.zeros((L,), jnp.int32)
    for p0 in range(0, PATHS, L):
        tot = zeros
        prior = zeros
        for w in range(NW):
            h = hist_all[pl.ds(w * PATHS + p0, L)]
            tot = tot + h
            prior = prior + jnp.where(
                jnp.broadcast_to(w < wid, (L,)), h, zeros)
        cnt_v[pl.ds(p0, L)] = tot
        base_v[pl.ds(p0, L)] = prior

    carry = jnp.int32(0)
    for p0 in range(0, PATHS, L):
        t = cnt_v[pl.ds(p0, L)]
        excl = plsc.cumsum(t) - t + carry
        base_v[pl.ds(p0, L)] = base_v[pl.ds(p0, L)] + excl
        carry = carry + jnp.sum(t)

    lane = lax.iota(jnp.int32, L)

    def grp(g, c):
        pid_g = pid_sl[pl.ds(g * L, L)]
        rk = rank_sl[pl.ds(g * L, L)]
        db = plsc.load_gather(base_v, [pid_g])
        plsc.store_scatter(dest2, [(L // CH) * g + lane // CH,
                                   lane & (CH - 1)], db + rk)
        return c

    lax.fori_loop(0, G, grp, 0)

    @pl.when(wid == 0)
    def _():
        pltpu.sync_copy(cnt_v, cnt_hbm)

    def chunk_src(c):
        return x_hbm.at[pl.ds(base + c * CH, CH)]

    NBUF = 3
    bufs = (rows_a, rows_b, rows_c)
    ld = [None] * NBUF
    st = [None] * NBUF
    for c in range(NBUF - 1):
        ld[c] = pltpu.async_copy(chunk_src(c), bufs[c], ld_sem)
    for c in range(NCH):
        b = c % NBUF
        ld[b].wait()
        st[b] = pltpu.async_copy(bufs[b], out_hbm.at[dest2.at[c]], st_sem)
        nc = c + NBUF - 1
        if nc < NCH:
            nb = nc % NBUF
            if st[nb] is not None:
                st[nb].wait()
            ld[nb] = pltpu.async_copy(chunk_src(nc), bufs[nb], ld_sem)
    for i in range(NBUF):
        st[(NCH - 1 - i) % NBUF].wait()


def kernel(inputs):
    n = inputs.shape[0]
    score = jax.random.normal(jax.random.key(1), (n * PATHS,),
                              dtype=jnp.float32)
    pid, rank, hist = _route_kernel(score)
    dispatched, counts = _dispatch_kernel(inputs, pid, rank, hist)
    return dispatched, pid, counts
